# baseline (speedup 1.0000x reference)
import functools

import jax
import jax.numpy as jnp
from jax import lax
from jax.experimental import pallas as pl
from jax.experimental.pallas import tpu as pltpu

D_MODEL = 1024
D_FF = 4096
E = 8
TOKENS = 4096
CAP = 640
SLOTS_PER_E = 8
NSLOT = E * SLOTS_PER_E
BLK = 512
NB = TOKENS // BLK
FBLK = 512
NF = D_FF // FBLK
LB_W = 0.01


def _router_select_kernel(x_ref, w_ref, xsel_ref, psel_ref, inv_ref, lb_ref,
                          cnt_ref, ps_ref):
    i = pl.program_id(0)
    xb = x_ref[...]
    logits = lax.dot_general(w_ref[...], xb, (((1,), (1,)), ((), ())),
                             preferred_element_type=jnp.float32)
    m = jnp.max(logits, axis=0, keepdims=True)
    ex = jnp.exp(logits - m)
    s = jnp.sum(ex, axis=0, keepdims=True)
    probs = ex / s
    iota_e = lax.broadcasted_iota(jnp.int32, (E, BLK), 0)
    idx = jnp.min(jnp.where(logits == m, iota_e, E), axis=0, keepdims=True)
    top1p = 1.0 / s

    oh = (iota_e == idx).astype(jnp.float32)
    counts_blk = jnp.sum(oh, axis=1, keepdims=True)
    probsum_blk = jnp.sum(probs, axis=1, keepdims=True)

    @pl.when(i == 0)
    def _init():
        cnt_ref[...] = jnp.zeros_like(cnt_ref)
        ps_ref[...] = jnp.zeros_like(ps_ref)
        xsel_ref[...] = jnp.zeros_like(xsel_ref)
        psel_ref[...] = jnp.zeros_like(psel_ref)

    prior = cnt_ref[...]
    lane = lax.broadcasted_iota(jnp.int32, (E, BLK), 1)
    c = oh
    k = 1
    while k < BLK:
        c = c + jnp.roll(c, k, axis=1) * (lane >= k).astype(jnp.float32)
        k *= 2
    rank_f = jnp.sum((prior + c) * oh, axis=0, keepdims=True) - 1.0
    rank = rank_f.astype(jnp.int32)
    sel = (rank % CAP) == 0
    slot = idx * SLOTS_PER_E + rank // CAP
    slot_oh = ((lax.broadcasted_iota(jnp.int32, (NSLOT, BLK), 0) == slot)
               .astype(jnp.float32) * sel.astype(jnp.float32))

    xsel_ref[...] += lax.dot_general(slot_oh, xb, (((1,), (0,)), ((), ())),
                                     preferred_element_type=jnp.float32)
    psel_ref[...] += lax.dot_general(slot_oh, top1p, (((1,), (1,)), ((), ())),
                                     preferred_element_type=jnp.float32)
    inv_ref[0] = jnp.where(sel, slot, NSLOT)

    cnt_ref[...] = prior + counts_blk
    ps_ref[...] = ps_ref[...] + probsum_blk

    @pl.when(i == NB - 1)
    def _finish():
        lb_ref[...] = (E * LB_W / (TOKENS * TOKENS)) * jnp.sum(
            cnt_ref[...] * ps_ref[...], keepdims=True)


def _ffn_kernel(xsel_ref, w1_ref, b1_ref, w2_ref, b2_ref, psel_ref, out_ref,
                acc_ref):
    f = pl.program_id(1)
    x8 = xsel_ref[...]
    pre = lax.dot_general(x8, w1_ref[0], (((1,), (1,)), ((), ())),
                          preferred_element_type=jnp.float32) + b1_ref[0]
    h = 0.5 * pre * (1.0 + lax.erf(pre * 0.7071067811865476))
    part = lax.dot_general(h, w2_ref[0], (((1,), (1,)), ((), ())),
                           preferred_element_type=jnp.float32)

    @pl.when(f == 0)
    def _init():
        acc_ref[...] = jnp.zeros_like(acc_ref)

    acc_ref[...] += part

    @pl.when(f == NF - 1)
    def _finish():
        out_ref[...] = (acc_ref[...] + b2_ref[0]) * psel_ref[...]


def _scatter_kernel(inv_ref, osel_ref, out_ref):
    oh = (lax.broadcasted_iota(jnp.int32, (NSLOT, BLK), 0)
          == inv_ref[0]).astype(jnp.float32)
    out_ref[...] = lax.dot_general(oh, osel_ref[...], (((0,), (0,)), ((), ())),
                                   preferred_element_type=jnp.float32)


@jax.jit
def kernel(x, router_W, fc1_W, fc1_b, fc2_W, fc2_b):
    B_, T_, D = x.shape
    x_flat = x.reshape(TOKENS, D)

    x_sel, p_sel, inv, lb = pl.pallas_call(
        _router_select_kernel,
        grid=(NB,),
        in_specs=[
            pl.BlockSpec((BLK, D_MODEL), lambda i: (i, 0)),
            pl.BlockSpec((E, D_MODEL), lambda i: (0, 0)),
        ],
        out_specs=[
            pl.BlockSpec((NSLOT, D_MODEL), lambda i: (0, 0)),
            pl.BlockSpec((NSLOT, 1), lambda i: (0, 0)),
            pl.BlockSpec((1, 1, BLK), lambda i: (i, 0, 0)),
            pl.BlockSpec((1, 1), lambda i: (0, 0)),
        ],
        out_shape=[
            jax.ShapeDtypeStruct((NSLOT, D_MODEL), jnp.float32),
            jax.ShapeDtypeStruct((NSLOT, 1), jnp.float32),
            jax.ShapeDtypeStruct((NB, 1, BLK), jnp.int32),
            jax.ShapeDtypeStruct((1, 1), jnp.float32),
        ],
        scratch_shapes=[
            pltpu.VMEM((E, 1), jnp.float32),
            pltpu.VMEM((E, 1), jnp.float32),
        ],
    )(x_flat, router_W)

    out_sel = pl.pallas_call(
        _ffn_kernel,
        grid=(E, NF),
        in_specs=[
            pl.BlockSpec((SLOTS_PER_E, D_MODEL), lambda e, f: (e, 0)),
            pl.BlockSpec((1, FBLK, D_MODEL), lambda e, f: (e, f, 0)),
            pl.BlockSpec((1, 1, FBLK), lambda e, f: (e, 0, f)),
            pl.BlockSpec((1, D_MODEL, FBLK), lambda e, f: (e, 0, f)),
            pl.BlockSpec((1, 1, D_MODEL), lambda e, f: (e, 0, 0)),
            pl.BlockSpec((SLOTS_PER_E, 1), lambda e, f: (e, 0)),
        ],
        out_specs=pl.BlockSpec((SLOTS_PER_E, D_MODEL), lambda e, f: (e, 0)),
        out_shape=jax.ShapeDtypeStruct((NSLOT, D_MODEL), jnp.float32),
        scratch_shapes=[pltpu.VMEM((SLOTS_PER_E, D_MODEL), jnp.float32)],
    )(x_sel, fc1_W, fc1_b.reshape(E, 1, D_FF), fc2_W,
      fc2_b.reshape(E, 1, D_MODEL), p_sel)

    out = pl.pallas_call(
        _scatter_kernel,
        grid=(NB,),
        in_specs=[
            pl.BlockSpec((1, 1, BLK), lambda i: (i, 0, 0)),
            pl.BlockSpec((NSLOT, D_MODEL), lambda i: (0, 0)),
        ],
        out_specs=pl.BlockSpec((BLK, D_MODEL), lambda i: (i, 0)),
        out_shape=jax.ShapeDtypeStruct((TOKENS, D_MODEL), jnp.float32),
    )(inv, out_sel)

    return out.reshape(B_, T_, D), lb.reshape(())


# speedup vs baseline: 1.1712x; 1.1712x over previous
import functools

import jax
import jax.numpy as jnp
from jax import lax
from jax.experimental import pallas as pl
from jax.experimental.pallas import tpu as pltpu

D_MODEL = 1024
D_FF = 4096
E = 8
TOKENS = 4096
CAP = 640
SLOTS_PER_E = 8
NSLOT = E * SLOTS_PER_E
BLK = 512
NB = TOKENS // BLK
FBLK = 1024
NF = D_FF // FBLK
LB_W = 0.01


def _router_select_kernel(x_ref, w_ref, xsel_ref, psel_ref, inv_ref, lb_ref,
                          cnt_ref, ps_ref):
    i = pl.program_id(0)
    xb = x_ref[...]
    logits = lax.dot_general(w_ref[...], xb, (((1,), (1,)), ((), ())),
                             preferred_element_type=jnp.float32)
    m = jnp.max(logits, axis=0, keepdims=True)
    ex = jnp.exp(logits - m)
    s = jnp.sum(ex, axis=0, keepdims=True)
    probs = ex / s
    iota_e = lax.broadcasted_iota(jnp.int32, (E, BLK), 0)
    idx = jnp.min(jnp.where(logits == m, iota_e, E), axis=0, keepdims=True)
    top1p = 1.0 / s

    oh = (iota_e == idx).astype(jnp.float32)
    counts_blk = jnp.sum(oh, axis=1, keepdims=True)
    probsum_blk = jnp.sum(probs, axis=1, keepdims=True)

    @pl.when(i == 0)
    def _init():
        cnt_ref[...] = jnp.zeros_like(cnt_ref)
        ps_ref[...] = jnp.zeros_like(ps_ref)
        xsel_ref[...] = jnp.zeros_like(xsel_ref)
        psel_ref[...] = jnp.zeros_like(psel_ref)

    prior = cnt_ref[...]
    lane = lax.broadcasted_iota(jnp.int32, (E, BLK), 1)
    c = oh
    k = 1
    while k < BLK:
        c = c + jnp.roll(c, k, axis=1) * (lane >= k).astype(jnp.float32)
        k *= 2
    rank_f = jnp.sum((prior + c) * oh, axis=0, keepdims=True) - 1.0
    rank = rank_f.astype(jnp.int32)
    sel = (rank % CAP) == 0
    slot = idx * SLOTS_PER_E + rank // CAP
    slot_oh = ((lax.broadcasted_iota(jnp.int32, (NSLOT, BLK), 0) == slot)
               .astype(jnp.float32) * sel.astype(jnp.float32))

    xsel_ref[...] += lax.dot_general(slot_oh, xb, (((1,), (0,)), ((), ())),
                                     preferred_element_type=jnp.float32)
    psel_ref[...] += lax.dot_general(slot_oh, top1p, (((1,), (1,)), ((), ())),
                                     preferred_element_type=jnp.float32)
    inv_ref[0] = jnp.where(sel, slot, NSLOT)

    cnt_ref[...] = prior + counts_blk
    ps_ref[...] = ps_ref[...] + probsum_blk

    @pl.when(i == NB - 1)
    def _finish():
        lb_ref[...] = (E * LB_W / (TOKENS * TOKENS)) * jnp.sum(
            cnt_ref[...] * ps_ref[...], keepdims=True)


def _ffn_kernel(xsel_ref, w1_ref, b1_ref, w2_ref, b2_ref, psel_ref, out_ref,
                acc_ref):
    f = pl.program_id(1)
    x8 = xsel_ref[...]
    pre = lax.dot_general(x8, w1_ref[0], (((1,), (1,)), ((), ())),
                          preferred_element_type=jnp.float32) + b1_ref[0]
    h = 0.5 * pre * (1.0 + lax.erf(pre * 0.7071067811865476))
    part = lax.dot_general(h, w2_ref[0], (((1,), (1,)), ((), ())),
                           preferred_element_type=jnp.float32)

    @pl.when(f == 0)
    def _init():
        acc_ref[...] = jnp.zeros_like(acc_ref)

    acc_ref[...] += part

    @pl.when(f == NF - 1)
    def _finish():
        out_ref[...] = (acc_ref[...] + b2_ref[0]) * psel_ref[...]


def _scatter_kernel(inv_ref, osel_ref, out_ref):
    oh = (lax.broadcasted_iota(jnp.int32, (NSLOT, BLK), 0)
          == inv_ref[0]).astype(jnp.float32)
    out_ref[...] = lax.dot_general(oh, osel_ref[...], (((0,), (0,)), ((), ())),
                                   preferred_element_type=jnp.float32)


@jax.jit
def kernel(x, router_W, fc1_W, fc1_b, fc2_W, fc2_b):
    B_, T_, D = x.shape
    x_flat = x.reshape(TOKENS, D)

    x_sel, p_sel, inv, lb = pl.pallas_call(
        _router_select_kernel,
        grid=(NB,),
        in_specs=[
            pl.BlockSpec((BLK, D_MODEL), lambda i: (i, 0)),
            pl.BlockSpec((E, D_MODEL), lambda i: (0, 0)),
        ],
        out_specs=[
            pl.BlockSpec((NSLOT, D_MODEL), lambda i: (0, 0)),
            pl.BlockSpec((NSLOT, 1), lambda i: (0, 0)),
            pl.BlockSpec((1, 1, BLK), lambda i: (i, 0, 0)),
            pl.BlockSpec((1, 1), lambda i: (0, 0)),
        ],
        out_shape=[
            jax.ShapeDtypeStruct((NSLOT, D_MODEL), jnp.float32),
            jax.ShapeDtypeStruct((NSLOT, 1), jnp.float32),
            jax.ShapeDtypeStruct((NB, 1, BLK), jnp.int32),
            jax.ShapeDtypeStruct((1, 1), jnp.float32),
        ],
        scratch_shapes=[
            pltpu.VMEM((E, 1), jnp.float32),
            pltpu.VMEM((E, 1), jnp.float32),
        ],
    )(x_flat, router_W)

    out_sel = pl.pallas_call(
        _ffn_kernel,
        grid=(E, NF),
        in_specs=[
            pl.BlockSpec((SLOTS_PER_E, D_MODEL), lambda e, f: (e, 0)),
            pl.BlockSpec((1, FBLK, D_MODEL), lambda e, f: (e, f, 0)),
            pl.BlockSpec((1, 1, FBLK), lambda e, f: (e, 0, f)),
            pl.BlockSpec((1, D_MODEL, FBLK), lambda e, f: (e, 0, f)),
            pl.BlockSpec((1, 1, D_MODEL), lambda e, f: (e, 0, 0)),
            pl.BlockSpec((SLOTS_PER_E, 1), lambda e, f: (e, 0)),
        ],
        out_specs=pl.BlockSpec((SLOTS_PER_E, D_MODEL), lambda e, f: (e, 0)),
        out_shape=jax.ShapeDtypeStruct((NSLOT, D_MODEL), jnp.float32),
        scratch_shapes=[pltpu.VMEM((SLOTS_PER_E, D_MODEL), jnp.float32)],
    )(x_sel, fc1_W, fc1_b.reshape(E, 1, D_FF), fc2_W,
      fc2_b.reshape(E, 1, D_MODEL), p_sel)

    out = pl.pallas_call(
        _scatter_kernel,
        grid=(NB,),
        in_specs=[
            pl.BlockSpec((1, 1, BLK), lambda i: (i, 0, 0)),
            pl.BlockSpec((NSLOT, D_MODEL), lambda i: (0, 0)),
        ],
        out_specs=pl.BlockSpec((BLK, D_MODEL), lambda i: (i, 0)),
        out_shape=jax.ShapeDtypeStruct((TOKENS, D_MODEL), jnp.float32),
    )(inv, out_sel)

    return out.reshape(B_, T_, D), lb.reshape(())
